# baseline (speedup 1.0000x reference)
import functools

import jax
import jax.numpy as jnp
from jax.experimental import pallas as pl
from jax.experimental.pallas import tpu as pltpu

_SB = 512


def _f32dot(a, b):
    return jax.lax.dot_general(a, b, (((1,), (0,)), ((), ())),
                               preferred_element_type=jnp.float32)


def _bdot(a, b):
    return jax.lax.dot_general(a.astype(jnp.bfloat16), b.astype(jnp.bfloat16),
                               (((1,), (0,)), ((), ())),
                               preferred_element_type=jnp.float32)


def _pass1_body(x_ref, wk_ref, wv_ref, pbd_ref, m_ref, psq_ref, itv_ref,
                ss_ref, norm_ref):
    c = pl.program_id(1)
    xb = x_ref[0]
    k = _f32dot(xb, wk_ref[...])
    v = _bdot(xb, wv_ref[...])
    kp = _f32dot(k, pbd_ref[...])
    k2s = _f32dot(k * k, m_ref[...])
    dk = k2s + psq_ref[...] - 2.0 * kp
    ak = jnp.exp(-jnp.maximum(dk, 0.0) * itv_ref[...])
    ssc = jax.lax.dot_general(ak.astype(jnp.bfloat16), v.astype(jnp.bfloat16),
                              (((0,), (0,)), ((), ())),
                              preferred_element_type=jnp.float32)
    nc = jnp.sum(ak, axis=0, keepdims=True)

    @pl.when(c == 0)
    def _():
        ss_ref[0] = ssc
        norm_ref[0] = nc

    @pl.when(c != 0)
    def _():
        ss_ref[0] += ssc
        norm_ref[0] += nc

    @pl.when(c == pl.num_programs(1) - 1)
    def _():
        ss_ref[0] = ss_ref[0] * m_ref[...]


def _pass2_body(x_ref, wq_ref, pbd_ref, m_ref, psq_ref, itv_ref, amp_ref,
                wo_ref, ss_ref, norm_ref, out_ref):
    xb = x_ref[0]
    q = _f32dot(xb, wq_ref[...])
    qp = _f32dot(q, pbd_ref[...])
    q2s = _f32dot(q * q, m_ref[...])
    dq = q2s + psq_ref[...] - 2.0 * qp
    w = jnp.exp(-jnp.maximum(dq, 0.0) * itv_ref[...]) * amp_ref[...]
    num = _bdot(w, ss_ref[0])
    den = _bdot(w * norm_ref[0], m_ref[...]) + 1e-8
    y = num / den
    out_ref[0] = _bdot(y, wo_ref[...])


def kernel(x, Wq, Wk, Wv, Wo, positions, log_scales, amplitudes):
    B, S, D = x.shape
    H, K, DH = positions.shape
    f32 = jnp.float32

    scales = jnp.exp(log_scales)
    itv = (0.5 / (scales * scales + 1e-6)).reshape(1, H * K)
    psq = jnp.sum(positions * positions, axis=-1).reshape(1, H * K)
    amp = amplitudes.reshape(1, H * K)
    eye_h = jnp.eye(H, dtype=f32)
    pbd = jnp.einsum('hg,hkd->hdgk', eye_h, positions).reshape(D, D)
    m = jnp.kron(eye_h, jnp.ones((DH, K), f32))

    nc = S // _SB
    grid = (B, nc)

    full = lambda b, c: (0, 0)
    xspec = pl.BlockSpec((1, _SB, D), lambda b, c: (b, c, 0))
    wspec = pl.BlockSpec((D, D), full)
    vspec = pl.BlockSpec((1, D), full)
    ss_spec = pl.BlockSpec((1, D, D), lambda b, c: (b, 0, 0))
    nm_spec = pl.BlockSpec((1, 1, D), lambda b, c: (b, 0, 0))

    ss, norm = pl.pallas_call(
        _pass1_body,
        grid=grid,
        in_specs=[xspec, wspec, wspec, wspec, wspec, vspec, vspec],
        out_specs=[ss_spec, nm_spec],
        out_shape=[jax.ShapeDtypeStruct((B, D, D), f32),
                   jax.ShapeDtypeStruct((B, 1, D), f32)],
        compiler_params=pltpu.CompilerParams(
            dimension_semantics=("arbitrary", "arbitrary")),
    )(x, Wk, Wv, pbd, m, psq, itv)

    out = pl.pallas_call(
        _pass2_body,
        grid=grid,
        in_specs=[xspec, wspec, wspec, wspec, vspec, vspec, vspec, wspec,
                  ss_spec, nm_spec],
        out_specs=xspec,
        out_shape=jax.ShapeDtypeStruct((B, S, D), f32),
        compiler_params=pltpu.CompilerParams(
            dimension_semantics=("parallel", "arbitrary")),
    )(x, Wq, pbd, m, psq, itv, amp, Wo, ss, norm)
    return out


# speedup vs baseline: 1.3701x; 1.3701x over previous
import jax
import jax.numpy as jnp
from jax.experimental import pallas as pl
from jax.experimental.pallas import tpu as pltpu

_SB = 512
_PAIR = 128


def _f32dot(a, b, ta=False):
    dims = (((0,) if ta else (1,), (0,)), ((), ()))
    return jax.lax.dot_general(a, b, dims, preferred_element_type=jnp.float32)


def _bdot(a, b, ta=False):
    dims = (((0,) if ta else (1,), (0,)), ((), ()))
    return jax.lax.dot_general(a.astype(jnp.bfloat16), b.astype(jnp.bfloat16),
                               dims, preferred_element_type=jnp.float32)


def _affinity(t, vcat_ref, psq_ref, itv_ref, npair):
    parts = []
    for c in range(npair):
        sl = slice(c * _PAIR, (c + 1) * _PAIR)
        tc = t[:, sl]
        g = _f32dot(jnp.concatenate([tc, tc * tc], axis=1), vcat_ref[c])
        parts.append(g + psq_ref[0, :, sl])
    d = jnp.concatenate(parts, axis=1)
    return jnp.exp(-jnp.maximum(d, 0.0) * itv_ref[0])


def _pass1_body(x_ref, wk_ref, wv_ref, vcat_ref, m2_ref, psq_ref, itv_ref,
                ss_ref, norm_ref):
    c = pl.program_id(1)
    npair = ss_ref.shape[1] // _PAIR
    xb = x_ref[0]
    k = _f32dot(xb, wk_ref[...])
    v = _bdot(xb, wv_ref[...])
    ak = _affinity(k, vcat_ref, psq_ref, itv_ref, npair)
    nc = jnp.sum(ak, axis=0, keepdims=True)

    akb = ak.astype(jnp.bfloat16)
    vb = v.astype(jnp.bfloat16)
    for p in range(npair):
        sl = slice(p * _PAIR, (p + 1) * _PAIR)
        blk = _f32dot(akb[:, sl], vb[:, sl], ta=True)

        @pl.when(c == 0)
        def _():
            ss_ref[0, sl, :] = blk

        @pl.when(c != 0)
        def _():
            ss_ref[0, sl, :] += blk

        @pl.when(c == pl.num_programs(1) - 1)
        def _():
            ss_ref[0, sl, :] = ss_ref[0, sl, :] * m2_ref[...]

    @pl.when(c == 0)
    def _():
        norm_ref[0] = nc

    @pl.when(c != 0)
    def _():
        norm_ref[0] += nc


def _pass2_body(x_ref, wq_ref, vcat_ref, psq_ref, itv_ref, amp_ref, wo_ref,
                ss_ref, dmat_ref, out_ref):
    npair = ss_ref.shape[1] // _PAIR
    xb = x_ref[0]
    q = _f32dot(xb, wq_ref[...])
    w = _affinity(q, vcat_ref, psq_ref, itv_ref, npair) * amp_ref[0]
    ys = []
    for p in range(npair):
        sl = slice(p * _PAIR, (p + 1) * _PAIR)
        rhs = jnp.concatenate([ss_ref[0, sl, :], dmat_ref[0, sl, :]], axis=1)
        nd = _bdot(w[:, sl], rhs)
        ys.append(nd[:, :_PAIR] / (nd[:, _PAIR:] + 1e-8))
    y = jnp.concatenate(ys, axis=1)
    out_ref[0] = _bdot(y, wo_ref[...])


def kernel(x, Wq, Wk, Wv, Wo, positions, log_scales, amplitudes):
    B, S, D = x.shape
    H, K, DH = positions.shape
    f32 = jnp.float32
    npair = H // 2

    scales = jnp.exp(log_scales)
    itv = (0.5 / (scales * scales + 1e-6)).reshape(1, 1, H * K)
    psq = jnp.sum(positions * positions, axis=-1).reshape(1, 1, H * K)
    amp = amplitudes.reshape(1, 1, H * K)
    eye_h = jnp.eye(H, dtype=f32)
    pbd = jnp.einsum('hg,hkd->hdgk', eye_h, positions).reshape(D, D)
    m2 = jnp.kron(jnp.eye(2, dtype=f32), jnp.ones((DH, K), f32))
    pbd_blocks = jnp.stack([pbd[c * _PAIR:(c + 1) * _PAIR,
                                c * _PAIR:(c + 1) * _PAIR]
                            for c in range(npair)])
    vcat = jnp.concatenate([-2.0 * pbd_blocks,
                            jnp.broadcast_to(m2, (npair, _PAIR, _PAIR))],
                           axis=1)

    nc = S // _SB
    grid = (B, nc)

    full = lambda b, c: (0, 0)
    full3 = lambda b, c: (0, 0, 0)
    xspec = pl.BlockSpec((1, _SB, D), lambda b, c: (b, c, 0))
    wspec = pl.BlockSpec((D, D), full)
    vcspec = pl.BlockSpec((npair, 2 * _PAIR, _PAIR), full3)
    rowspec = pl.BlockSpec((1, 1, D), full3)
    m2spec = pl.BlockSpec((_PAIR, _PAIR), full)
    ss_spec = pl.BlockSpec((1, D, _PAIR), lambda b, c: (b, 0, 0))
    nm_spec = pl.BlockSpec((1, 1, D), lambda b, c: (b, 0, 0))

    ss, norm = pl.pallas_call(
        _pass1_body,
        grid=grid,
        in_specs=[xspec, wspec, wspec, vcspec, m2spec, rowspec, rowspec],
        out_specs=[ss_spec, nm_spec],
        out_shape=[jax.ShapeDtypeStruct((B, D, _PAIR), f32),
                   jax.ShapeDtypeStruct((B, 1, D), f32)],
        compiler_params=pltpu.CompilerParams(
            dimension_semantics=("arbitrary", "arbitrary")),
    )(x, Wk, Wv, vcat, m2, psq, itv)

    dmat = norm[:, 0, :, None] * jnp.tile(m2, (npair, 1))[None]

    out = pl.pallas_call(
        _pass2_body,
        grid=grid,
        in_specs=[xspec, wspec, vcspec, rowspec, rowspec, rowspec, wspec,
                  ss_spec, ss_spec],
        out_specs=xspec,
        out_shape=jax.ShapeDtypeStruct((B, S, D), f32),
        compiler_params=pltpu.CompilerParams(
            dimension_semantics=("parallel", "arbitrary")),
    )(x, Wq, vcat, psq, itv, amp, Wo, ss, dmat)
    return out


# speedup vs baseline: 1.5506x; 1.1317x over previous
import jax
import jax.numpy as jnp
from jax.experimental import pallas as pl
from jax.experimental.pallas import tpu as pltpu

_SB = 1024
_PAIR = 128


def _f32dot(a, b, ta=False):
    dims = (((0,) if ta else (1,), (0,)), ((), ()))
    return jax.lax.dot_general(a, b, dims, preferred_element_type=jnp.float32)


def _bdot(a, b, ta=False):
    dims = (((0,) if ta else (1,), (0,)), ((), ()))
    return jax.lax.dot_general(a.astype(jnp.bfloat16), b.astype(jnp.bfloat16),
                               dims, preferred_element_type=jnp.float32)


def _affinity(t, vcat_ref, psq_ref, itv_ref, npair):
    parts = []
    for c in range(npair):
        sl = slice(c * _PAIR, (c + 1) * _PAIR)
        tc = t[:, sl]
        g = _f32dot(jnp.concatenate([tc, tc * tc], axis=1), vcat_ref[c])
        parts.append(g + psq_ref[0, :, sl])
    d = jnp.concatenate(parts, axis=1)
    return jnp.exp(-jnp.maximum(d, 0.0) * itv_ref[0])


def _pass1_body(x_ref, wk_ref, wv_ref, vcat_ref, m2_ref, psq_ref, itv_ref,
                ss_ref, norm_ref):
    c = pl.program_id(1)
    npair = ss_ref.shape[1] // _PAIR
    xb = x_ref[0]
    k = _f32dot(xb, wk_ref[...])
    v = _bdot(xb, wv_ref[...])
    ak = _affinity(k, vcat_ref, psq_ref, itv_ref, npair)
    nc = _f32dot(jnp.ones((8, ak.shape[0]), jnp.float32), ak)[0:1]

    akb = ak.astype(jnp.bfloat16)
    vb = v.astype(jnp.bfloat16)
    for p in range(npair):
        sl = slice(p * _PAIR, (p + 1) * _PAIR)
        blk = _f32dot(akb[:, sl], vb[:, sl], ta=True)

        @pl.when(c == 0)
        def _():
            ss_ref[0, sl, :] = blk

        @pl.when(c != 0)
        def _():
            ss_ref[0, sl, :] += blk

        @pl.when(c == pl.num_programs(1) - 1)
        def _():
            ss_ref[0, sl, :] = ss_ref[0, sl, :] * m2_ref[...]

    @pl.when(c == 0)
    def _():
        norm_ref[0] = nc

    @pl.when(c != 0)
    def _():
        norm_ref[0] += nc


def _pass2_body(x_ref, wq_ref, vcat_ref, psq_ref, itv_ref, amp_ref, wo_ref,
                ss_ref, dmat_ref, out_ref):
    npair = ss_ref.shape[1] // _PAIR
    xb = x_ref[0]
    q = _f32dot(xb, wq_ref[...])
    w = _affinity(q, vcat_ref, psq_ref, itv_ref, npair) * amp_ref[0]
    ys = []
    for p in range(npair):
        sl = slice(p * _PAIR, (p + 1) * _PAIR)
        rhs = jnp.concatenate([ss_ref[0, sl, :], dmat_ref[0, sl, :]], axis=1)
        nd = _bdot(w[:, sl], rhs)
        ys.append(nd[:, :_PAIR] / (nd[:, _PAIR:] + 1e-8))
    y = jnp.concatenate(ys, axis=1)
    out_ref[0] = _bdot(y, wo_ref[...])


def kernel(x, Wq, Wk, Wv, Wo, positions, log_scales, amplitudes):
    B, S, D = x.shape
    H, K, DH = positions.shape
    f32 = jnp.float32
    npair = H // 2

    scales = jnp.exp(log_scales)
    itv = (0.5 / (scales * scales + 1e-6)).reshape(1, 1, H * K)
    psq = jnp.sum(positions * positions, axis=-1).reshape(1, 1, H * K)
    amp = amplitudes.reshape(1, 1, H * K)
    eye_h = jnp.eye(H, dtype=f32)
    pbd = jnp.einsum('hg,hkd->hdgk', eye_h, positions).reshape(D, D)
    m2 = jnp.kron(jnp.eye(2, dtype=f32), jnp.ones((DH, K), f32))
    pbd_blocks = jnp.stack([pbd[c * _PAIR:(c + 1) * _PAIR,
                                c * _PAIR:(c + 1) * _PAIR]
                            for c in range(npair)])
    vcat = jnp.concatenate([-2.0 * pbd_blocks,
                            jnp.broadcast_to(m2, (npair, _PAIR, _PAIR))],
                           axis=1)

    nc = S // _SB
    grid = (B, nc)

    full = lambda b, c: (0, 0)
    full3 = lambda b, c: (0, 0, 0)
    xspec = pl.BlockSpec((1, _SB, D), lambda b, c: (b, c, 0))
    wspec = pl.BlockSpec((D, D), full)
    vcspec = pl.BlockSpec((npair, 2 * _PAIR, _PAIR), full3)
    rowspec = pl.BlockSpec((1, 1, D), full3)
    m2spec = pl.BlockSpec((_PAIR, _PAIR), full)
    ss_spec = pl.BlockSpec((1, D, _PAIR), lambda b, c: (b, 0, 0))
    nm_spec = pl.BlockSpec((1, 1, D), lambda b, c: (b, 0, 0))

    ss, norm = pl.pallas_call(
        _pass1_body,
        grid=grid,
        in_specs=[xspec, wspec, wspec, vcspec, m2spec, rowspec, rowspec],
        out_specs=[ss_spec, nm_spec],
        out_shape=[jax.ShapeDtypeStruct((B, D, _PAIR), f32),
                   jax.ShapeDtypeStruct((B, 1, D), f32)],
        compiler_params=pltpu.CompilerParams(
            dimension_semantics=("arbitrary", "arbitrary")),
    )(x, Wk, Wv, vcat, m2, psq, itv)

    dmat = norm[:, 0, :, None] * jnp.tile(m2, (npair, 1))[None]

    out = pl.pallas_call(
        _pass2_body,
        grid=grid,
        in_specs=[xspec, wspec, vcspec, rowspec, rowspec, rowspec, wspec,
                  ss_spec, ss_spec],
        out_specs=xspec,
        out_shape=jax.ShapeDtypeStruct((B, S, D), f32),
        compiler_params=pltpu.CompilerParams(
            dimension_semantics=("parallel", "arbitrary")),
    )(x, Wq, vcat, psq, itv, amp, Wo, ss, dmat)
    return out
